# baseline (speedup 1.0000x reference)
import jax
import jax.numpy as jnp
from jax import lax
from jax.experimental import pallas as pl
from jax.experimental.pallas import tpu as pltpu
from jax.experimental.pallas import tpu_sc as plsc

N = 10000
E = 320000
F0 = 128
FH = 128
G = 64
NCORES = 2
NSUB = 16
LANES = 16
CHUNK = 128
NPAD = 10240
DUMMY_DST = 10200
ROWS_PER_TILE = NPAD // NSUB
ZCHUNKS = ROWS_PER_TILE // CHUNK
LAST_ROWS = N - (NSUB - 1) * ROWS_PER_TILE

_DO_GATHER = True
_DO_SCATTER = True
NSPLIT = 2
IB = 16
NC1 = 80
NC2 = 160


import functools


@functools.lru_cache(maxsize=None)
def _make_segsum(n_chunks):
    mesh = plsc.VectorSubcoreMesh(
        core_axis_name="c", subcore_axis_name="s",
        num_cores=NCORES, num_subcores=NSUB)
    out_t = (jax.ShapeDtypeStruct((N, FH), jnp.float32),
             jax.ShapeDtypeStruct((N, FH), jnp.float32))
    nb = n_chunks // IB
    scratch = [
        pltpu.VMEM((2, IB, CHUNK), jnp.int32),
        pltpu.VMEM((2, IB, CHUNK), jnp.int32),
        pltpu.VMEM((2, CHUNK, FH), jnp.float32),
        pltpu.VMEM_SHARED((NPAD, FH), jnp.float32),
        pltpu.SemaphoreType.DMA,
        pltpu.SemaphoreType.DMA,
        pltpu.SemaphoreType.DMA,
    ]

    def body(t0, t1, sidx, didx, o0, o1, sbuf, dbuf, rbuf, acc,
             gsem0, gsem1, isem):
        c = lax.axis_index("c")
        s = lax.axis_index("s")
        gsems = (gsem0, gsem1)

        def fire_gather(idx_row, slot):
            if not _DO_GATHER:
                return

            @pl.when(c == 0)
            def _():
                for h in range(NSPLIT):
                    hw = CHUNK // NSPLIT
                    pltpu.async_copy(t0.at[idx_row.at[pl.ds(h * hw, hw)]],
                                     rbuf.at[slot, pl.ds(h * hw, hw)],
                                     gsems[slot])

            @pl.when(c == 1)
            def _():
                for h in range(NSPLIT):
                    hw = CHUNK // NSPLIT
                    pltpu.async_copy(t1.at[idx_row.at[pl.ds(h * hw, hw)]],
                                     rbuf.at[slot, pl.ds(h * hw, hw)],
                                     gsems[slot])

        def wait_gather(slot):
            if not _DO_GATHER:
                return
            pltpu.make_async_copy(
                t0.at[sbuf.at[0, 0]], rbuf.at[slot], gsems[slot]).wait()

        def zrow(i, carry):
            for k in range(FH // LANES):
                rbuf[0, i, pl.ds(k * LANES, LANES)] = jnp.zeros(
                    (LANES,), jnp.float32)
            return carry
        lax.fori_loop(0, CHUNK, zrow, 0)
        for k in range(ZCHUNKS):
            pltpu.sync_copy(
                rbuf.at[0], acc.at[pl.ds(s * ROWS_PER_TILE + k * CHUNK, CHUNK)])
        plsc.subcore_barrier()

        pltpu.sync_copy(sidx.at[c, s, pl.ds(0, IB)], sbuf.at[0])
        pltpu.sync_copy(didx.at[c, s, pl.ds(0, IB)], dbuf.at[0])
        fire_gather(sbuf.at[0, 0], 0)

        def block_body(b, carry):
            nxt = b + 1
            pb = lax.rem(b, 2)
            pn = lax.rem(nxt, 2)

            @pl.when(nxt < nb)
            def _():
                pltpu.async_copy(sidx.at[c, s, pl.ds(nxt * IB, IB)],
                                 sbuf.at[pn], isem)
                pltpu.async_copy(didx.at[c, s, pl.ds(nxt * IB, IB)],
                                 dbuf.at[pn], isem)

            for k in range(IB):
                cur = k % 2
                wait_gather(cur)
                if k + 1 < IB:
                    fire_gather(sbuf.at[pb, k + 1], (k + 1) % 2)
                else:
                    @pl.when(nxt < nb)
                    def _():
                        pltpu.make_async_copy(
                            sidx.at[c, s, pl.ds(0, IB)], sbuf.at[pn],
                            isem).wait()
                        pltpu.make_async_copy(
                            didx.at[c, s, pl.ds(0, IB)], dbuf.at[pn],
                            isem).wait()
                        fire_gather(sbuf.at[pn, 0], 0)
                if _DO_SCATTER:
                    pltpu.sync_copy(rbuf.at[cur], acc.at[dbuf.at[pb, k]],
                                    add=True)
            return carry
        lax.fori_loop(0, nb, block_body, 0)
        plsc.subcore_barrier()

        row0 = s * ROWS_PER_TILE
        last0 = (NSUB - 1) * ROWS_PER_TILE

        @pl.when(c == 0)
        def _():
            @pl.when(s < NSUB - 1)
            def _():
                pltpu.sync_copy(acc.at[pl.ds(row0, ROWS_PER_TILE)],
                                o0.at[pl.ds(row0, ROWS_PER_TILE)])

            @pl.when(s == NSUB - 1)
            def _():
                pltpu.sync_copy(acc.at[pl.ds(last0, LAST_ROWS)],
                                o0.at[pl.ds(last0, LAST_ROWS)])

        @pl.when(c == 1)
        def _():
            @pl.when(s < NSUB - 1)
            def _():
                pltpu.sync_copy(acc.at[pl.ds(row0, ROWS_PER_TILE)],
                                o1.at[pl.ds(row0, ROWS_PER_TILE)])

            @pl.when(s == NSUB - 1)
            def _():
                pltpu.sync_copy(acc.at[pl.ds(last0, LAST_ROWS)],
                                o1.at[pl.ds(last0, LAST_ROWS)])

    return pl.kernel(body, out_type=out_t, mesh=mesh, scratch_types=scratch)


def _mlp1_body(x_r, p0_r, p1_r, wa_r, ba_r, g_r, be_r, wb_r, bb_r, oa_r, ob_r):
    h = x_r[...] + p0_r[...] + p1_r[...]
    hp = jnp.dot(h, wa_r[...], preferred_element_type=jnp.float32) + ba_r[...]
    mu = jnp.mean(hp, axis=0, keepdims=True)
    var = jnp.mean(hp * hp, axis=0, keepdims=True) - mu * mu
    hn = (hp - mu) * (g_r[...] * lax.rsqrt(var + 1e-5)) + be_r[...]
    hn = jnp.maximum(hn, 0.0)
    h1 = jnp.maximum(
        jnp.dot(hn, wb_r[...], preferred_element_type=jnp.float32) + bb_r[...],
        0.0)
    oa_r[...] = h1[:, :FH]
    ob_r[...] = h1[:, FH:]


_mlp1 = pl.pallas_call(
    _mlp1_body,
    out_shape=(jax.ShapeDtypeStruct((N, FH), jnp.float32),
               jax.ShapeDtypeStruct((N, FH), jnp.float32)))


def _mlp2_body(ha_r, hb_r, qa_r, qb_r, b_r, wa_r, ba_r, g_r, be_r, wb_r, bb_r,
               wl_r, bl_r, o_r):
    h = jnp.concatenate([ha_r[...] + qa_r[...], hb_r[...] + qb_r[...]], axis=1)
    hp = jnp.dot(h, wa_r[...], preferred_element_type=jnp.float32) + ba_r[...]
    mu = jnp.mean(hp, axis=0, keepdims=True)
    var = jnp.mean(hp * hp, axis=0, keepdims=True) - mu * mu
    hn = (hp - mu) * (g_r[...] * lax.rsqrt(var + 1e-5)) + be_r[...]
    hn = jnp.maximum(hn, 0.0)
    h2 = jnp.maximum(
        jnp.dot(hn, wb_r[...], preferred_element_type=jnp.float32) + bb_r[...],
        0.0)
    gid = lax.broadcasted_iota(jnp.int32, (G, N), 0)
    onehot = (b_r[...] == gid).astype(jnp.float32)
    sums = jnp.dot(onehot, h2, preferred_element_type=jnp.float32)
    counts = jnp.sum(onehot, axis=1, keepdims=True)
    pooled = sums / jnp.maximum(counts, 1.0)
    o_r[...] = (jnp.dot(pooled, wl_r[...], preferred_element_type=jnp.float32)
                + bl_r[...])


_mlp2 = pl.pallas_call(
    _mlp2_body,
    out_shape=jax.ShapeDtypeStruct((G, 256), jnp.float32))


def kernel(x, adj, batch, W1a, b1a, g1, be1, W1b, b1b,
           W2a, b2a, g2, be2, W2b, b2b, Wl, bl):
    src = adj[0].astype(jnp.int32)
    dst = adj[1].astype(jnp.int32)

    half = E // 2
    pad1 = NSUB * NC1 * CHUNK - half
    s1 = jnp.pad(src.reshape(2, half), ((0, 0), (0, pad1)),
                 constant_values=0).reshape(2, NSUB, NC1, CHUNK)
    d1 = jnp.pad(dst.reshape(2, half), ((0, 0), (0, pad1)),
                 constant_values=DUMMY_DST).reshape(2, NSUB, NC1, CHUNK)
    p0, p1 = _make_segsum(NC1)(x, x, s1, d1)
    h1a, h1b = _mlp1(x, p0, p1, W1a, b1a.reshape(1, -1), g1.reshape(1, -1),
                     be1.reshape(1, -1), W1b, b1b.reshape(1, -1))

    pad2 = NSUB * NC2 * CHUNK - E
    s2 = jnp.broadcast_to(
        jnp.pad(src, (0, pad2), constant_values=0
                ).reshape(1, NSUB, NC2, CHUNK),
        (2, NSUB, NC2, CHUNK))
    d2 = jnp.broadcast_to(
        jnp.pad(dst, (0, pad2), constant_values=DUMMY_DST
                ).reshape(1, NSUB, NC2, CHUNK),
        (2, NSUB, NC2, CHUNK))
    qa, qb = _make_segsum(NC2)(h1a, h1b, s2, d2)

    out = _mlp2(h1a, h1b, qa, qb, batch.astype(jnp.int32).reshape(1, N),
                W2a, b2a.reshape(1, -1), g2.reshape(1, -1), be2.reshape(1, -1),
                W2b, b2b.reshape(1, -1), Wl, bl.reshape(1, -1))
    return out


# speedup vs baseline: 1.9077x; 1.9077x over previous
import functools

import jax
import jax.numpy as jnp
from jax import lax
from jax.experimental import pallas as pl
from jax.experimental.pallas import tpu as pltpu
from jax.experimental.pallas import tpu_sc as plsc

N = 10000
E = 320000
F0 = 128
FQ = 64
G = 64
NCORES = 2
NSUB = 16
LANES = 16
CHUNK = 128
NPAD = 10240
DUMMY_DST = 10200
ROWS_PER_TILE = NPAD // NSUB
ZCHUNKS = ROWS_PER_TILE // CHUNK
LAST_ROWS = N - (NSUB - 1) * ROWS_PER_TILE
TROWS = N // NSUB
IB = 16
NC = 160
NB = NC // IB


@functools.lru_cache(maxsize=None)
def _make_segsum():
    mesh = plsc.VectorSubcoreMesh(
        core_axis_name="c", subcore_axis_name="s",
        num_cores=NCORES, num_subcores=NSUB)
    out_t = (jax.ShapeDtypeStruct((N, FQ), jnp.float32),
             jax.ShapeDtypeStruct((N, FQ), jnp.float32))
    scratch = [
        pltpu.VMEM((2, IB, CHUNK), jnp.int32),
        pltpu.VMEM((2, IB, CHUNK), jnp.int32),
        pltpu.VMEM((2, CHUNK, FQ), jnp.float32),
        pltpu.VMEM_SHARED((N, FQ), jnp.float32),
        pltpu.VMEM_SHARED((NPAD, FQ), jnp.float32),
        pltpu.SemaphoreType.DMA,
        pltpu.SemaphoreType.DMA,
        pltpu.SemaphoreType.DMA,
    ]

    def body(t0, t1, sidx, didx, o0, o1, sbuf, dbuf, rbuf, tab, acc,
             gsem0, gsem1, isem):
        c = lax.axis_index("c")
        s = lax.axis_index("s")
        gsems = (gsem0, gsem1)

        def fire_gather(idx_row, slot):
            pltpu.async_copy(tab.at[idx_row], rbuf.at[slot], gsems[slot])

        def wait_gather(slot):
            pltpu.make_async_copy(
                t0.at[pl.ds(0, CHUNK)], rbuf.at[slot], gsems[slot]).wait()

        trow = s * ROWS_PER_TILE
        tlast = (NSUB - 1) * ROWS_PER_TILE

        @pl.when(c == 0)
        def _():
            @pl.when(s < NSUB - 1)
            def _():
                pltpu.sync_copy(t0.at[pl.ds(trow, ROWS_PER_TILE)],
                                tab.at[pl.ds(trow, ROWS_PER_TILE)])

            @pl.when(s == NSUB - 1)
            def _():
                pltpu.sync_copy(t0.at[pl.ds(tlast, LAST_ROWS)],
                                tab.at[pl.ds(tlast, LAST_ROWS)])

        @pl.when(c == 1)
        def _():
            @pl.when(s < NSUB - 1)
            def _():
                pltpu.sync_copy(t1.at[pl.ds(trow, ROWS_PER_TILE)],
                                tab.at[pl.ds(trow, ROWS_PER_TILE)])

            @pl.when(s == NSUB - 1)
            def _():
                pltpu.sync_copy(t1.at[pl.ds(tlast, LAST_ROWS)],
                                tab.at[pl.ds(tlast, LAST_ROWS)])

        def zrow(i, carry):
            for k in range(FQ // LANES):
                rbuf[0, i, pl.ds(k * LANES, LANES)] = jnp.zeros(
                    (LANES,), jnp.float32)
            return carry
        lax.fori_loop(0, CHUNK, zrow, 0)
        for k in range(ZCHUNKS):
            pltpu.sync_copy(
                rbuf.at[0], acc.at[pl.ds(s * ROWS_PER_TILE + k * CHUNK, CHUNK)])
        plsc.subcore_barrier()

        pltpu.sync_copy(sidx.at[s, pl.ds(0, IB)], sbuf.at[0])
        pltpu.sync_copy(didx.at[s, pl.ds(0, IB)], dbuf.at[0])
        fire_gather(sbuf.at[0, 0], 0)

        def block_body(b, carry):
            nxt = b + 1
            pb = lax.rem(b, 2)
            pn = lax.rem(nxt, 2)

            @pl.when(nxt < NB)
            def _():
                pltpu.async_copy(sidx.at[s, pl.ds(nxt * IB, IB)],
                                 sbuf.at[pn], isem)
                pltpu.async_copy(didx.at[s, pl.ds(nxt * IB, IB)],
                                 dbuf.at[pn], isem)

            for k in range(IB):
                cur = k % 2
                wait_gather(cur)
                if k + 1 < IB:
                    fire_gather(sbuf.at[pb, k + 1], (k + 1) % 2)
                else:
                    @pl.when(nxt < NB)
                    def _():
                        pltpu.make_async_copy(
                            sidx.at[s, pl.ds(0, IB)], sbuf.at[pn],
                            isem).wait()
                        pltpu.make_async_copy(
                            didx.at[s, pl.ds(0, IB)], dbuf.at[pn],
                            isem).wait()
                        fire_gather(sbuf.at[pn, 0], 0)
                pltpu.sync_copy(rbuf.at[cur], acc.at[dbuf.at[pb, k]], add=True)
            return carry
        lax.fori_loop(0, NB, block_body, 0)
        plsc.subcore_barrier()

        row0 = s * ROWS_PER_TILE
        last0 = (NSUB - 1) * ROWS_PER_TILE

        @pl.when(c == 0)
        def _():
            @pl.when(s < NSUB - 1)
            def _():
                pltpu.sync_copy(acc.at[pl.ds(row0, ROWS_PER_TILE)],
                                o0.at[pl.ds(row0, ROWS_PER_TILE)])

            @pl.when(s == NSUB - 1)
            def _():
                pltpu.sync_copy(acc.at[pl.ds(last0, LAST_ROWS)],
                                o0.at[pl.ds(last0, LAST_ROWS)])

        @pl.when(c == 1)
        def _():
            @pl.when(s < NSUB - 1)
            def _():
                pltpu.sync_copy(acc.at[pl.ds(row0, ROWS_PER_TILE)],
                                o1.at[pl.ds(row0, ROWS_PER_TILE)])

            @pl.when(s == NSUB - 1)
            def _():
                pltpu.sync_copy(acc.at[pl.ds(last0, LAST_ROWS)],
                                o1.at[pl.ds(last0, LAST_ROWS)])

    return pl.kernel(
        body, out_type=out_t, mesh=mesh, scratch_types=scratch,
        compiler_params=pltpu.CompilerParams(use_tc_tiling_on_sc=False))


def _mlp1_body(x_r, a0_r, a1_r, wa_r, ba_r, g_r, be_r, wb_r, bb_r,
               o0_r, o1_r, o2_r, o3_r):
    h = x_r[...] + jnp.concatenate([a0_r[...], a1_r[...]], axis=1)
    hp = jnp.dot(h, wa_r[...], preferred_element_type=jnp.float32) + ba_r[...]
    mu = jnp.mean(hp, axis=0, keepdims=True)
    var = jnp.mean(hp * hp, axis=0, keepdims=True) - mu * mu
    hn = (hp - mu) * (g_r[...] * lax.rsqrt(var + 1e-5)) + be_r[...]
    hn = jnp.maximum(hn, 0.0)
    h1 = jnp.maximum(
        jnp.dot(hn, wb_r[...], preferred_element_type=jnp.float32) + bb_r[...],
        0.0)
    o0_r[...] = h1[:, 0 * FQ:1 * FQ]
    o1_r[...] = h1[:, 1 * FQ:2 * FQ]
    o2_r[...] = h1[:, 2 * FQ:3 * FQ]
    o3_r[...] = h1[:, 3 * FQ:4 * FQ]


_mlp1 = pl.pallas_call(
    _mlp1_body,
    out_shape=tuple(jax.ShapeDtypeStruct((N, FQ), jnp.float32)
                    for _ in range(4)))


def _mlp2_body(h0_r, h1_r, h2_r, h3_r, a0_r, a1_r, a2_r, a3_r, b_r,
               wa_r, ba_r, g_r, be_r, wb_r, bb_r, wl_r, bl_r, o_r):
    h = jnp.concatenate([h0_r[...] + a0_r[...], h1_r[...] + a1_r[...],
                         h2_r[...] + a2_r[...], h3_r[...] + a3_r[...]], axis=1)
    hp = jnp.dot(h, wa_r[...], preferred_element_type=jnp.float32) + ba_r[...]
    mu = jnp.mean(hp, axis=0, keepdims=True)
    var = jnp.mean(hp * hp, axis=0, keepdims=True) - mu * mu
    hn = (hp - mu) * (g_r[...] * lax.rsqrt(var + 1e-5)) + be_r[...]
    hn = jnp.maximum(hn, 0.0)
    h2 = jnp.maximum(
        jnp.dot(hn, wb_r[...], preferred_element_type=jnp.float32) + bb_r[...],
        0.0)
    gid = lax.broadcasted_iota(jnp.int32, (G, N), 0)
    onehot = (b_r[...] == gid).astype(jnp.float32)
    sums = jnp.dot(onehot, h2, preferred_element_type=jnp.float32)
    counts = jnp.sum(onehot, axis=1, keepdims=True)
    pooled = sums / jnp.maximum(counts, 1.0)
    o_r[...] = (jnp.dot(pooled, wl_r[...], preferred_element_type=jnp.float32)
                + bl_r[...])


_mlp2 = pl.pallas_call(
    _mlp2_body,
    out_shape=jax.ShapeDtypeStruct((G, 256), jnp.float32))


def kernel(x, adj, batch, W1a, b1a, g1, be1, W1b, b1b,
           W2a, b2a, g2, be2, W2b, b2b, Wl, bl):
    src = adj[0].astype(jnp.int32)
    dst = adj[1].astype(jnp.int32)
    pad = NSUB * NC * CHUNK - E
    sidx = jnp.pad(src, (0, pad), constant_values=0).reshape(NSUB, NC, CHUNK)
    didx = jnp.pad(dst, (0, pad),
                   constant_values=DUMMY_DST).reshape(NSUB, NC, CHUNK)

    seg = _make_segsum()
    a1a, a1b = seg(x[:, :FQ], x[:, FQ:], sidx, didx)
    q0, q1, q2, q3 = _mlp1(x, a1a, a1b, W1a, b1a.reshape(1, -1),
                           g1.reshape(1, -1), be1.reshape(1, -1), W1b,
                           b1b.reshape(1, -1))
    aq0, aq1 = seg(q0, q1, sidx, didx)
    aq2, aq3 = seg(q2, q3, sidx, didx)

    out = _mlp2(q0, q1, q2, q3, aq0, aq1, aq2, aq3,
                batch.astype(jnp.int32).reshape(1, N),
                W2a, b2a.reshape(1, -1), g2.reshape(1, -1), be2.reshape(1, -1),
                W2b, b2b.reshape(1, -1), Wl, bl.reshape(1, -1))
    return out


# speedup vs baseline: 1.9463x; 1.0203x over previous
import functools

import jax
import jax.numpy as jnp
from jax import lax
from jax.experimental import pallas as pl
from jax.experimental.pallas import tpu as pltpu
from jax.experimental.pallas import tpu_sc as plsc

N = 10000
E = 320000
F0 = 128
FQ = 64
G = 64
NCORES = 2
NSUB = 16
LANES = 16
CHUNK = 256
NPAD = 10240
DUMMY_DST = 10200
ROWS_PER_TILE = NPAD // NSUB
ZROWS = 128
ZCHUNKS = ROWS_PER_TILE // ZROWS
LAST_ROWS = N - (NSUB - 1) * ROWS_PER_TILE
TROWS = N // NSUB
IB = 8
NC = 80
NB = NC // IB


@functools.lru_cache(maxsize=None)
def _make_segsum():
    mesh = plsc.VectorSubcoreMesh(
        core_axis_name="c", subcore_axis_name="s",
        num_cores=NCORES, num_subcores=NSUB)
    out_t = (jax.ShapeDtypeStruct((N, FQ), jnp.float32),
             jax.ShapeDtypeStruct((N, FQ), jnp.float32))
    scratch = [
        pltpu.VMEM((2, IB, CHUNK), jnp.int32),
        pltpu.VMEM((2, IB, CHUNK), jnp.int32),
        pltpu.VMEM((2, CHUNK, FQ), jnp.float32),
        pltpu.VMEM_SHARED((N, FQ), jnp.float32),
        pltpu.VMEM_SHARED((NPAD, FQ), jnp.float32),
        pltpu.SemaphoreType.DMA,
        pltpu.SemaphoreType.DMA,
        pltpu.SemaphoreType.DMA,
    ]

    def body(t0, t1, sidx, didx, o0, o1, sbuf, dbuf, rbuf, tab, acc,
             gsem0, gsem1, isem):
        c = lax.axis_index("c")
        s = lax.axis_index("s")
        gsems = (gsem0, gsem1)

        def fire_gather(idx_row, slot):
            pltpu.async_copy(tab.at[idx_row], rbuf.at[slot], gsems[slot])

        def wait_gather(slot):
            pltpu.make_async_copy(
                t0.at[pl.ds(0, CHUNK)], rbuf.at[slot], gsems[slot]).wait()

        trow = s * ROWS_PER_TILE
        tlast = (NSUB - 1) * ROWS_PER_TILE

        @pl.when(c == 0)
        def _():
            @pl.when(s < NSUB - 1)
            def _():
                pltpu.sync_copy(t0.at[pl.ds(trow, ROWS_PER_TILE)],
                                tab.at[pl.ds(trow, ROWS_PER_TILE)])

            @pl.when(s == NSUB - 1)
            def _():
                pltpu.sync_copy(t0.at[pl.ds(tlast, LAST_ROWS)],
                                tab.at[pl.ds(tlast, LAST_ROWS)])

        @pl.when(c == 1)
        def _():
            @pl.when(s < NSUB - 1)
            def _():
                pltpu.sync_copy(t1.at[pl.ds(trow, ROWS_PER_TILE)],
                                tab.at[pl.ds(trow, ROWS_PER_TILE)])

            @pl.when(s == NSUB - 1)
            def _():
                pltpu.sync_copy(t1.at[pl.ds(tlast, LAST_ROWS)],
                                tab.at[pl.ds(tlast, LAST_ROWS)])

        def zrow(i, carry):
            for k in range(FQ // LANES):
                rbuf[0, i, pl.ds(k * LANES, LANES)] = jnp.zeros(
                    (LANES,), jnp.float32)
            return carry
        lax.fori_loop(0, CHUNK, zrow, 0)
        for k in range(ZCHUNKS):
            pltpu.sync_copy(
                rbuf.at[0, pl.ds(0, ZROWS)],
                acc.at[pl.ds(s * ROWS_PER_TILE + k * ZROWS, ZROWS)])
        plsc.subcore_barrier()

        pltpu.sync_copy(sidx.at[s, pl.ds(0, IB)], sbuf.at[0])
        pltpu.sync_copy(didx.at[s, pl.ds(0, IB)], dbuf.at[0])
        fire_gather(sbuf.at[0, 0], 0)

        def block_body(b, carry):
            nxt = b + 1
            pb = lax.rem(b, 2)
            pn = lax.rem(nxt, 2)

            @pl.when(nxt < NB)
            def _():
                pltpu.async_copy(sidx.at[s, pl.ds(nxt * IB, IB)],
                                 sbuf.at[pn], isem)
                pltpu.async_copy(didx.at[s, pl.ds(nxt * IB, IB)],
                                 dbuf.at[pn], isem)

            for k in range(IB):
                cur = k % 2
                wait_gather(cur)
                if k + 1 < IB:
                    fire_gather(sbuf.at[pb, k + 1], (k + 1) % 2)
                else:
                    @pl.when(nxt < NB)
                    def _():
                        pltpu.make_async_copy(
                            sidx.at[s, pl.ds(0, IB)], sbuf.at[pn],
                            isem).wait()
                        pltpu.make_async_copy(
                            didx.at[s, pl.ds(0, IB)], dbuf.at[pn],
                            isem).wait()
                        fire_gather(sbuf.at[pn, 0], 0)
                pltpu.sync_copy(rbuf.at[cur], acc.at[dbuf.at[pb, k]], add=True)
            return carry
        lax.fori_loop(0, NB, block_body, 0)
        plsc.subcore_barrier()

        row0 = s * ROWS_PER_TILE
        last0 = (NSUB - 1) * ROWS_PER_TILE

        @pl.when(c == 0)
        def _():
            @pl.when(s < NSUB - 1)
            def _():
                pltpu.sync_copy(acc.at[pl.ds(row0, ROWS_PER_TILE)],
                                o0.at[pl.ds(row0, ROWS_PER_TILE)])

            @pl.when(s == NSUB - 1)
            def _():
                pltpu.sync_copy(acc.at[pl.ds(last0, LAST_ROWS)],
                                o0.at[pl.ds(last0, LAST_ROWS)])

        @pl.when(c == 1)
        def _():
            @pl.when(s < NSUB - 1)
            def _():
                pltpu.sync_copy(acc.at[pl.ds(row0, ROWS_PER_TILE)],
                                o1.at[pl.ds(row0, ROWS_PER_TILE)])

            @pl.when(s == NSUB - 1)
            def _():
                pltpu.sync_copy(acc.at[pl.ds(last0, LAST_ROWS)],
                                o1.at[pl.ds(last0, LAST_ROWS)])

    return pl.kernel(
        body, out_type=out_t, mesh=mesh, scratch_types=scratch,
        compiler_params=pltpu.CompilerParams(use_tc_tiling_on_sc=False))


def _mlp1_body(x_r, a0_r, a1_r, wa_r, ba_r, g_r, be_r, wb_r, bb_r,
               o0_r, o1_r, o2_r, o3_r):
    h = x_r[...] + jnp.concatenate([a0_r[...], a1_r[...]], axis=1)
    hp = jnp.dot(h, wa_r[...], preferred_element_type=jnp.float32) + ba_r[...]
    mu = jnp.mean(hp, axis=0, keepdims=True)
    var = jnp.mean(hp * hp, axis=0, keepdims=True) - mu * mu
    hn = (hp - mu) * (g_r[...] * lax.rsqrt(var + 1e-5)) + be_r[...]
    hn = jnp.maximum(hn, 0.0)
    h1 = jnp.maximum(
        jnp.dot(hn, wb_r[...], preferred_element_type=jnp.float32) + bb_r[...],
        0.0)
    o0_r[...] = h1[:, 0 * FQ:1 * FQ]
    o1_r[...] = h1[:, 1 * FQ:2 * FQ]
    o2_r[...] = h1[:, 2 * FQ:3 * FQ]
    o3_r[...] = h1[:, 3 * FQ:4 * FQ]


_mlp1 = pl.pallas_call(
    _mlp1_body,
    out_shape=tuple(jax.ShapeDtypeStruct((N, FQ), jnp.float32)
                    for _ in range(4)))


def _mlp2_body(h0_r, h1_r, h2_r, h3_r, a0_r, a1_r, a2_r, a3_r, b_r,
               wa_r, ba_r, g_r, be_r, wb_r, bb_r, wl_r, bl_r, o_r):
    h = jnp.concatenate([h0_r[...] + a0_r[...], h1_r[...] + a1_r[...],
                         h2_r[...] + a2_r[...], h3_r[...] + a3_r[...]], axis=1)
    hp = jnp.dot(h, wa_r[...], preferred_element_type=jnp.float32) + ba_r[...]
    mu = jnp.mean(hp, axis=0, keepdims=True)
    var = jnp.mean(hp * hp, axis=0, keepdims=True) - mu * mu
    hn = (hp - mu) * (g_r[...] * lax.rsqrt(var + 1e-5)) + be_r[...]
    hn = jnp.maximum(hn, 0.0)
    h2 = jnp.maximum(
        jnp.dot(hn, wb_r[...], preferred_element_type=jnp.float32) + bb_r[...],
        0.0)
    gid = lax.broadcasted_iota(jnp.int32, (G, N), 0)
    onehot = (b_r[...] == gid).astype(jnp.float32)
    sums = jnp.dot(onehot, h2, preferred_element_type=jnp.float32)
    counts = jnp.sum(onehot, axis=1, keepdims=True)
    pooled = sums / jnp.maximum(counts, 1.0)
    o_r[...] = (jnp.dot(pooled, wl_r[...], preferred_element_type=jnp.float32)
                + bl_r[...])


_mlp2 = pl.pallas_call(
    _mlp2_body,
    out_shape=jax.ShapeDtypeStruct((G, 256), jnp.float32))


def kernel(x, adj, batch, W1a, b1a, g1, be1, W1b, b1b,
           W2a, b2a, g2, be2, W2b, b2b, Wl, bl):
    src = adj[0].astype(jnp.int32)
    dst = adj[1].astype(jnp.int32)
    pad = NSUB * NC * CHUNK - E
    sidx = jnp.pad(src, (0, pad), constant_values=0).reshape(NSUB, NC, CHUNK)
    didx = jnp.pad(dst, (0, pad),
                   constant_values=DUMMY_DST).reshape(NSUB, NC, CHUNK)

    seg = _make_segsum()
    a1a, a1b = seg(x[:, :FQ], x[:, FQ:], sidx, didx)
    q0, q1, q2, q3 = _mlp1(x, a1a, a1b, W1a, b1a.reshape(1, -1),
                           g1.reshape(1, -1), be1.reshape(1, -1), W1b,
                           b1b.reshape(1, -1))
    aq0, aq1 = seg(q0, q1, sidx, didx)
    aq2, aq3 = seg(q2, q3, sidx, didx)

    out = _mlp2(q0, q1, q2, q3, aq0, aq1, aq2, aq3,
                batch.astype(jnp.int32).reshape(1, N),
                W2a, b2a.reshape(1, -1), g2.reshape(1, -1), be2.reshape(1, -1),
                W2b, b2b.reshape(1, -1), Wl, bl.reshape(1, -1))
    return out
